# Initial kernel scaffold; baseline (speedup 1.0000x reference)
import jax
import jax.numpy as jnp
from jax.experimental import pallas as pl


def kernel(logits, targets):
    raise NotImplementedError("write your pallas kernel here")


# speedup vs baseline: 5.8971x; 5.8971x over previous
import functools

import jax
import jax.numpy as jnp
from jax import lax
from jax.experimental import pallas as pl
from jax.experimental.pallas import tpu as pltpu
from jax.experimental.pallas import tpu_sc as plsc

_LANES = 16


@functools.lru_cache(maxsize=None)
def _build(B, C, K):
    NC, NS = 2, 16
    NW = NC * NS
    RW = B // NW
    NG = RW // _LANES
    OUTW = _LANES

    mesh = plsc.VectorSubcoreMesh(core_axis_name="c", subcore_axis_name="s")

    @functools.partial(
        pl.kernel,
        mesh=mesh,
        out_type=jax.ShapeDtypeStruct((B * OUTW,), jnp.float32),
        compiler_params=pltpu.CompilerParams(needs_layout_passes=False),
        scratch_types=[
            pltpu.VMEM((_LANES * C,), jnp.float32),
            pltpu.VMEM((RW,), jnp.int32),
            pltpu.VMEM((RW * OUTW,), jnp.float32),
        ],
    )
    def sc_mine(logits_hbm, targets_hbm, out_hbm, buf, tgt_v, out_v):
        wid = lax.axis_index("s") * NC + lax.axis_index("c")
        row0 = wid * RW
        pltpu.sync_copy(targets_hbm.at[pl.ds(row0, RW)], tgt_v)

        iota = lax.iota(jnp.int32, _LANES)
        gidx = iota * C
        ninf = jnp.full((_LANES,), -jnp.inf, jnp.float32)

        for g in range(NG):
            pltpu.sync_copy(
                logits_hbm.at[pl.ds((row0 + g * _LANES) * C, _LANES * C)],
                buf,
            )
            tgt = tgt_v[pl.ds(g * _LANES, _LANES)]

            def body(c, carry, tgt=tgt):
                pos, tops = carry
                cvec = jnp.full((_LANES,), c, jnp.int32)
                v = plsc.load_gather(buf, [gidx + cvec])
                eq = tgt == cvec
                pos = jnp.where(eq, v, pos)
                new = jnp.where(eq, ninf, v)
                upd = []
                for t in tops:
                    hi = jnp.maximum(t, new)
                    new = jnp.minimum(t, new)
                    upd.append(hi)
                return pos, tuple(upd)

            pos, tops = lax.fori_loop(0, C, body, (ninf, (ninf,) * K))

            base = g * (_LANES * OUTW) + iota * OUTW
            plsc.store_scatter(out_v, [base], pos)
            for j in range(1, K):
                plsc.store_scatter(out_v, [base + j], tops[j])
            for j in range(K, OUTW):
                plsc.store_scatter(out_v, [base + j], ninf)

        pltpu.sync_copy(out_v, out_hbm.at[pl.ds(row0 * OUTW, RW * OUTW)])

    def tc_finish(res_ref, out_ref):
        x = res_ref[...] * 10.0
        m = jnp.max(x, axis=1, keepdims=True)
        s = jnp.sum(jnp.exp(x - m), axis=1)
        lse = m[:, 0] + jnp.log(s)
        loss = lse - x[:, 0]
        out_ref[...] = (jnp.sum(loss) * (1.0 / B)).reshape(1, 1)

    tc_call = pl.pallas_call(
        tc_finish,
        out_shape=jax.ShapeDtypeStruct((1, 1), jnp.float32),
    )

    def run(logits, targets):
        res = sc_mine(logits.reshape(-1), targets)
        return tc_call(res.reshape(B, OUTW))[0, 0]

    return run


def kernel(logits, targets):
    B, C = logits.shape
    K = int(0.01 * (C - 1))
    return _build(B, C, K)(logits, targets.astype(jnp.int32))


# speedup vs baseline: 6.2929x; 1.0671x over previous
import functools

import jax
import jax.numpy as jnp
from jax import lax
from jax.experimental import pallas as pl
from jax.experimental.pallas import tpu as pltpu
from jax.experimental.pallas import tpu_sc as plsc

_LANES = 16


@functools.lru_cache(maxsize=None)
def _build(B, C, K):
    NC, NS = 2, 16
    NW = NC * NS
    RW = B // NW
    NG = RW // _LANES
    OUTW = _LANES

    mesh = plsc.VectorSubcoreMesh(core_axis_name="c", subcore_axis_name="s")

    GPI = 2
    NP = NG // GPI

    @functools.partial(
        pl.kernel,
        mesh=mesh,
        out_type=jax.ShapeDtypeStruct((B * OUTW,), jnp.float32),
        compiler_params=pltpu.CompilerParams(needs_layout_passes=False),
        scratch_types=[
            pltpu.VMEM((GPI * _LANES * C,), jnp.float32),
            pltpu.VMEM((RW,), jnp.int32),
            pltpu.VMEM((RW * OUTW,), jnp.float32),
        ],
    )
    def sc_mine(logits_hbm, targets_hbm, out_hbm, buf, tgt_v, out_v):
        wid = lax.axis_index("s") * NC + lax.axis_index("c")
        row0 = wid * RW
        pltpu.sync_copy(targets_hbm.at[pl.ds(row0, RW)], tgt_v)

        iota = lax.iota(jnp.int32, _LANES)
        gidx = iota * C
        ninf = jnp.full((_LANES,), -jnp.inf, jnp.float32)
        one = jnp.full((_LANES,), 1, jnp.int32)
        cmax = jnp.full((_LANES,), C, jnp.int32)
        phase = (iota * 9) & 15

        for p in range(NP):
            pltpu.sync_copy(
                logits_hbm.at[
                    pl.ds((row0 + p * GPI * _LANES) * C, GPI * _LANES * C)
                ],
                buf,
            )
            tgts = [
                tgt_v[pl.ds((p * GPI + i) * _LANES, _LANES)]
                for i in range(GPI)
            ]

            def body(_, carry, tgts=tgts):
                colv, poss, topss = carry
                new_poss, new_topss = [], []
                for i in range(GPI):
                    addr = gidx + colv + (i * _LANES * C)
                    v = plsc.load_gather(buf, [addr])
                    eq = tgts[i] == colv
                    new_poss.append(jnp.where(eq, v, poss[i]))
                    new = jnp.where(eq, ninf, v)
                    upd = []
                    for t in topss[i]:
                        hi = jnp.maximum(t, new)
                        new = jnp.minimum(t, new)
                        upd.append(hi)
                    new_topss.append(tuple(upd))
                colv = colv + one
                colv = jnp.where(colv == cmax, 0, colv)
                return colv, tuple(new_poss), tuple(new_topss)

            _, poss, topss = lax.fori_loop(
                0, C, body,
                (phase, (ninf,) * GPI, ((ninf,) * K,) * GPI),
            )

            for i in range(GPI):
                g = p * GPI + i
                base = g * (_LANES * OUTW) + iota * OUTW
                plsc.store_scatter(out_v, [base], poss[i])
                for j in range(1, K):
                    plsc.store_scatter(out_v, [base + j], topss[i][j])
                for j in range(K, OUTW):
                    plsc.store_scatter(out_v, [base + j], ninf)

        pltpu.sync_copy(out_v, out_hbm.at[pl.ds(row0 * OUTW, RW * OUTW)])

    def tc_finish(res_ref, out_ref):
        x = res_ref[...] * 10.0
        m = jnp.max(x, axis=1, keepdims=True)
        s = jnp.sum(jnp.exp(x - m), axis=1)
        lse = m[:, 0] + jnp.log(s)
        loss = lse - x[:, 0]
        out_ref[...] = (jnp.sum(loss) * (1.0 / B)).reshape(1, 1)

    tc_call = pl.pallas_call(
        tc_finish,
        out_shape=jax.ShapeDtypeStruct((1, 1), jnp.float32),
    )

    def run(logits, targets):
        res = sc_mine(logits.reshape(-1), targets)
        return tc_call(res.reshape(B, OUTW))[0, 0]

    return run


def kernel(logits, targets):
    B, C = logits.shape
    K = int(0.01 * (C - 1))
    return _build(B, C, K)(logits, targets.astype(jnp.int32))


# speedup vs baseline: 7.4617x; 1.1857x over previous
import functools

import jax
import jax.numpy as jnp
from jax import lax
from jax.experimental import pallas as pl
from jax.experimental.pallas import tpu as pltpu
from jax.experimental.pallas import tpu_sc as plsc

_LANES = 16


@functools.lru_cache(maxsize=None)
def _build(B, C, K):
    NC, NS = 2, 16
    NW = NC * NS
    RW = B // NW
    NG = RW // _LANES
    OUTW = _LANES

    mesh = plsc.VectorSubcoreMesh(core_axis_name="c", subcore_axis_name="s")

    GPI = 2
    NP = NG // GPI

    @functools.partial(
        pl.kernel,
        mesh=mesh,
        out_type=jax.ShapeDtypeStruct((B, OUTW), jnp.float32),
        compiler_params=pltpu.CompilerParams(needs_layout_passes=False),
        scratch_types=[
            pltpu.VMEM((GPI * _LANES, C), jnp.float32),
            pltpu.VMEM((RW,), jnp.int32),
            pltpu.VMEM((RW, OUTW), jnp.float32),
        ],
    )
    def sc_mine(logits_hbm, targets_hbm, out_hbm, buf, tgt_v, out_v):
        wid = lax.axis_index("s") * NC + lax.axis_index("c")
        row0 = wid * RW
        pltpu.sync_copy(targets_hbm.at[pl.ds(row0, RW)], tgt_v)

        iota = lax.iota(jnp.int32, _LANES)
        ninf = jnp.full((_LANES,), -jnp.inf, jnp.float32)
        one = jnp.full((_LANES,), 1, jnp.int32)
        cmax = jnp.full((_LANES,), C, jnp.int32)
        phase = (iota * 9) & 15

        for p in range(NP):
            pltpu.sync_copy(
                logits_hbm.at[pl.ds(row0 + p * GPI * _LANES, GPI * _LANES), :],
                buf,
            )
            tgts = [
                tgt_v[pl.ds((p * GPI + i) * _LANES, _LANES)]
                for i in range(GPI)
            ]

            def body(_, carry, tgts=tgts):
                colv, poss, topss = carry
                new_poss, new_topss = [], []
                for i in range(GPI):
                    v = plsc.load_gather(buf, [iota + i * _LANES, colv])
                    eq = tgts[i] == colv
                    new_poss.append(jnp.where(eq, v, poss[i]))
                    new = jnp.where(eq, ninf, v)
                    upd = []
                    for t in topss[i]:
                        hi = jnp.maximum(t, new)
                        new = jnp.minimum(t, new)
                        upd.append(hi)
                    new_topss.append(tuple(upd))
                colv = colv + one
                colv = jnp.where(colv == cmax, 0, colv)
                return colv, tuple(new_poss), tuple(new_topss)

            _, poss, topss = lax.fori_loop(
                0, C, body,
                (phase, (ninf,) * GPI, ((ninf,) * K,) * GPI),
            )

            for i in range(GPI):
                rows = (p * GPI + i) * _LANES + iota
                plsc.store_scatter(out_v, [rows, jnp.full((_LANES,), 0, jnp.int32)], poss[i])
                for j in range(1, K):
                    plsc.store_scatter(out_v, [rows, jnp.full((_LANES,), j, jnp.int32)], topss[i][j])
                for j in range(K, OUTW):
                    plsc.store_scatter(out_v, [rows, jnp.full((_LANES,), j, jnp.int32)], ninf)

        pltpu.sync_copy(out_v, out_hbm.at[pl.ds(row0, RW), :])

    def tc_finish(res_ref, out_ref):
        x = res_ref[...] * 10.0
        m = jnp.max(x, axis=1, keepdims=True)
        s = jnp.sum(jnp.exp(x - m), axis=1)
        lse = m[:, 0] + jnp.log(s)
        loss = lse - x[:, 0]
        out_ref[...] = (jnp.sum(loss) * (1.0 / B)).reshape(1, 1)

    tc_call = pl.pallas_call(
        tc_finish,
        out_shape=jax.ShapeDtypeStruct((1, 1), jnp.float32),
    )

    def run(logits, targets):
        res = sc_mine(logits, targets)
        return tc_call(res)[0, 0]

    return run


def kernel(logits, targets):
    B, C = logits.shape
    K = int(0.01 * (C - 1))
    return _build(B, C, K)(logits, targets.astype(jnp.int32))


# speedup vs baseline: 8.3305x; 1.1164x over previous
import functools

import jax
import jax.numpy as jnp
from jax import lax
from jax.experimental import pallas as pl
from jax.experimental.pallas import tpu as pltpu
from jax.experimental.pallas import tpu_sc as plsc

_LANES = 16


@functools.lru_cache(maxsize=None)
def _build(B, C, K):
    NC, NS = 2, 16
    NW = NC * NS
    RW = B // NW
    NG = RW // _LANES
    OUTW = _LANES

    mesh = plsc.VectorSubcoreMesh(core_axis_name="c", subcore_axis_name="s")

    GPI = 4
    NP = NG // GPI

    @functools.partial(
        pl.kernel,
        mesh=mesh,
        out_type=jax.ShapeDtypeStruct((B, OUTW), jnp.float32),
        compiler_params=pltpu.CompilerParams(needs_layout_passes=False),
        scratch_types=[
            pltpu.VMEM((GPI * _LANES, C), jnp.float32),
            pltpu.VMEM((RW,), jnp.int32),
            pltpu.VMEM((RW, OUTW), jnp.float32),
        ],
    )
    def sc_mine(logits_hbm, targets_hbm, out_hbm, buf, tgt_v, out_v):
        wid = lax.axis_index("s") * NC + lax.axis_index("c")
        row0 = wid * RW
        pltpu.sync_copy(targets_hbm.at[pl.ds(row0, RW)], tgt_v)

        iota = lax.iota(jnp.int32, _LANES)
        ninf = jnp.full((_LANES,), -jnp.inf, jnp.float32)
        one = jnp.full((_LANES,), 1, jnp.int32)
        cmax = jnp.full((_LANES,), C, jnp.int32)
        phase = (iota * 9) & 15

        for p in range(NP):
            pltpu.sync_copy(
                logits_hbm.at[pl.ds(row0 + p * GPI * _LANES, GPI * _LANES), :],
                buf,
            )
            tgts = [
                tgt_v[pl.ds((p * GPI + i) * _LANES, _LANES)]
                for i in range(GPI)
            ]

            def body(_, carry, tgts=tgts):
                colv, poss, topss = carry
                new_poss, new_topss = [], []
                for i in range(GPI):
                    v = plsc.load_gather(buf, [iota + i * _LANES, colv])
                    eq = tgts[i] == colv
                    new_poss.append(jnp.where(eq, v, poss[i]))
                    new = jnp.where(eq, ninf, v)
                    upd = []
                    for t in topss[i]:
                        hi = jnp.maximum(t, new)
                        new = jnp.minimum(t, new)
                        upd.append(hi)
                    new_topss.append(tuple(upd))
                colv = colv + one
                colv = jnp.where(colv == cmax, 0, colv)
                return colv, tuple(new_poss), tuple(new_topss)

            _, poss, topss = lax.fori_loop(
                0, C, body,
                (phase, (ninf,) * GPI, ((ninf,) * K,) * GPI),
            )

            for i in range(GPI):
                rows = (p * GPI + i) * _LANES + iota
                plsc.store_scatter(out_v, [rows, jnp.full((_LANES,), 0, jnp.int32)], poss[i])
                for j in range(1, K):
                    plsc.store_scatter(out_v, [rows, jnp.full((_LANES,), j, jnp.int32)], topss[i][j])
                for j in range(K, OUTW):
                    plsc.store_scatter(out_v, [rows, jnp.full((_LANES,), j, jnp.int32)], ninf)

        pltpu.sync_copy(out_v, out_hbm.at[pl.ds(row0, RW), :])

    def tc_finish(res_ref, out_ref):
        x = res_ref[...] * 10.0
        m = jnp.max(x, axis=1, keepdims=True)
        s = jnp.sum(jnp.exp(x - m), axis=1)
        lse = m[:, 0] + jnp.log(s)
        loss = lse - x[:, 0]
        out_ref[...] = (jnp.sum(loss) * (1.0 / B)).reshape(1, 1)

    tc_call = pl.pallas_call(
        tc_finish,
        out_shape=jax.ShapeDtypeStruct((1, 1), jnp.float32),
    )

    def run(logits, targets):
        res = sc_mine(logits, targets)
        return tc_call(res)[0, 0]

    return run


def kernel(logits, targets):
    B, C = logits.shape
    K = int(0.01 * (C - 1))
    return _build(B, C, K)(logits, targets.astype(jnp.int32))
